# Initial kernel scaffold; baseline (speedup 1.0000x reference)
import jax
import jax.numpy as jnp
from jax.experimental import pallas as pl


def kernel(x, node_ids, node_types, node_scores, edge_index, edge_type, edge_attr, node2graph, W_x2h, b_x2h, W_nt, b_nt, W_ns, b_ns, W_h2h, b_h2h, W_e1, b_e1, W_e2, b_e2, W_gat, W_gat_e, att_src, att_dst, att_edge, b_gat, W_m1, b_m1, W_m2, b_m2):
    raise NotImplementedError("write your pallas kernel here")


# speedup vs baseline: 5.5320x; 5.5320x over previous
import functools
import jax
import jax.numpy as jnp
from jax import lax
from jax.experimental import pallas as pl
from jax.experimental.pallas import tpu as pltpu, tpu_sc as plsc

N = 10000
E = 320000
G = 32
HID = 128
CHUNK = 2000
NSTEP = CHUNK // 16
NCHUNK = E // CHUNK


def _tc1_body(x, nt, ns, W_x2h, b_x2h, W_nt, b_nt, W_ns, b_ns,
              Wh_a, Wh_b, Wh_c, b_h2h, W_gat, att_s, att_d,
              xt_o, as_o, ad_o):
    f32 = jnp.float32
    h1 = jnp.dot(x[...], W_x2h[...], preferred_element_type=f32) + b_x2h[...]
    ntE = jnp.dot(nt[...], W_nt[...], preferred_element_type=f32) + b_nt[...]
    nsE = ns[...] * W_ns[...] + b_ns[...]
    h = (jnp.dot(h1, Wh_a[...], preferred_element_type=f32)
         + jnp.dot(ntE, Wh_b[...], preferred_element_type=f32)
         + jnp.dot(nsE, Wh_c[...], preferred_element_type=f32)
         + b_h2h[...])
    h = jnp.maximum(h, 0.0)
    xt = jnp.dot(h, W_gat[...], preferred_element_type=f32)
    xt_o[...] = xt
    as_o[...] = jnp.sum(xt * att_s[...], axis=1, keepdims=True)
    ad_o[...] = jnp.sum(xt * att_d[...], axis=1, keepdims=True)


def _tc1(x, node_types, node_scores, W_x2h, b_x2h, W_nt, b_nt, W_ns, b_ns,
         W_h2h, b_h2h, W_gat, att_src, att_dst):
    BN = 400
    grid = (N // BN,)
    row = lambda i: (i, 0)
    zero = lambda i: (0, 0)
    in_specs = [
        pl.BlockSpec((BN, 128), row),
        pl.BlockSpec((BN, 4), row),
        pl.BlockSpec((BN, 1), row),
        pl.BlockSpec((128, HID), zero),
        pl.BlockSpec((1, HID), zero),
        pl.BlockSpec((4, HID // 2), zero),
        pl.BlockSpec((1, HID // 2), zero),
        pl.BlockSpec((1, HID // 2), zero),
        pl.BlockSpec((1, HID // 2), zero),
        pl.BlockSpec((HID, HID), zero),
        pl.BlockSpec((HID // 2, HID), zero),
        pl.BlockSpec((HID // 2, HID), zero),
        pl.BlockSpec((1, HID), zero),
        pl.BlockSpec((HID, HID), zero),
        pl.BlockSpec((1, HID), zero),
        pl.BlockSpec((1, HID), zero),
    ]
    out_specs = [
        pl.BlockSpec((BN, HID), row),
        pl.BlockSpec((BN, 1), row),
        pl.BlockSpec((BN, 1), row),
    ]
    out_shape = [
        jax.ShapeDtypeStruct((N, HID), jnp.float32),
        jax.ShapeDtypeStruct((N, 1), jnp.float32),
        jax.ShapeDtypeStruct((N, 1), jnp.float32),
    ]
    return pl.pallas_call(
        _tc1_body, grid=grid, in_specs=in_specs, out_specs=out_specs,
        out_shape=out_shape,
    )(x, node_types, node_scores,
      W_x2h, b_x2h.reshape(1, -1), W_nt, b_nt.reshape(1, -1),
      W_ns, b_ns.reshape(1, -1),
      W_h2h[:HID], W_h2h[HID:HID + HID // 2], W_h2h[HID + HID // 2:],
      b_h2h.reshape(1, -1), W_gat,
      att_src.reshape(1, -1), att_dst.reshape(1, -1))


def _tc2_body(ea, W_e1, b_e1, W_e2, b_e2, W_gat_e, att_e, ae_o):
    f32 = jnp.float32
    h1 = jnp.maximum(jnp.dot(ea[...], W_e1[...], preferred_element_type=f32)
                     + b_e1[...], 0.0)
    h2 = jnp.maximum(jnp.dot(h1, W_e2[...], preferred_element_type=f32)
                     + b_e2[...], 0.0)
    v = jnp.dot(W_gat_e[...], att_e[...].reshape(HID, 1),
                preferred_element_type=f32)
    ae_o[...] = jnp.dot(h2, v, preferred_element_type=f32)


def _tc2(edge_attr, W_e1, b_e1, W_e2, b_e2, W_gat_e, att_edge):
    BE = 3200
    grid = (E // BE,)
    row = lambda i: (i, 0)
    zero = lambda i: (0, 0)
    in_specs = [
        pl.BlockSpec((BE, 46), row),
        pl.BlockSpec((46, HID), zero),
        pl.BlockSpec((1, HID), zero),
        pl.BlockSpec((HID, HID), zero),
        pl.BlockSpec((1, HID), zero),
        pl.BlockSpec((HID, HID), zero),
        pl.BlockSpec((1, HID), zero),
    ]
    return pl.pallas_call(
        _tc2_body, grid=grid, in_specs=in_specs,
        out_specs=pl.BlockSpec((BE, 1), row),
        out_shape=jax.ShapeDtypeStruct((E, 1), jnp.float32),
    )(edge_attr, W_e1, b_e1.reshape(1, -1), W_e2, b_e2.reshape(1, -1),
      W_gat_e, att_edge.reshape(1, -1))


def _sc_body(src_h, dst_h, ae_h, as_h, ad_h, n2g_h, c_out,
             as_v, ad_v, n2g_v, den_v, c_v, src_v, dst_v, ae_v):
    wid = lax.axis_index("s") * 2 + lax.axis_index("c")

    pltpu.sync_copy(as_h, as_v)
    pltpu.sync_copy(ad_h, ad_v)
    pltpu.sync_copy(n2g_h, n2g_v)

    def zero_body(i, carry):
        den_v[pl.ds(i * 16, 16)] = jnp.zeros((16,), jnp.float32)
        c_v[pl.ds(i * 16, 16)] = jnp.zeros((16,), jnp.float32)
        return carry
    lax.fori_loop(0, N // 16, zero_body, 0)

    def edge_vals(j):
        s16 = src_v[pl.ds(j * 16, 16)]
        d16 = dst_v[pl.ds(j * 16, 16)]
        ae16 = ae_v[pl.ds(j * 16, 16)]
        a = plsc.load_gather(as_v, [s16]) + plsc.load_gather(ad_v, [d16]) + ae16
        a = jnp.maximum(a, 0.2 * a)
        ex = jnp.exp(a)
        g16 = plsc.load_gather(n2g_v, [d16])
        return s16, d16, ex, g16 == wid

    def passA_chunk(ci, carry):
        pltpu.sync_copy(src_h.at[pl.ds(ci * CHUNK, CHUNK)], src_v)
        pltpu.sync_copy(dst_h.at[pl.ds(ci * CHUNK, CHUNK)], dst_v)
        pltpu.sync_copy(ae_h.at[pl.ds(ci * CHUNK, CHUNK)], ae_v)

        def step(j, c2):
            _, d16, ex, m = edge_vals(j)
            plsc.addupdate_scatter(den_v, [d16], ex, mask=m)
            return c2
        return lax.fori_loop(0, NSTEP, step, carry)
    lax.fori_loop(0, NCHUNK, passA_chunk, 0)

    def passB_chunk(ci, carry):
        pltpu.sync_copy(src_h.at[pl.ds(ci * CHUNK, CHUNK)], src_v)
        pltpu.sync_copy(dst_h.at[pl.ds(ci * CHUNK, CHUNK)], dst_v)
        pltpu.sync_copy(ae_h.at[pl.ds(ci * CHUNK, CHUNK)], ae_v)

        def step(j, c2):
            s16, d16, ex, m = edge_vals(j)
            den = plsc.load_gather(den_v, [d16])
            w = ex / (den + 1e-16)
            plsc.addupdate_scatter(c_v, [s16], w, mask=m)
            return c2
        return lax.fori_loop(0, NSTEP, step, carry)
    lax.fori_loop(0, NCHUNK, passB_chunk, 0)

    pltpu.sync_copy(c_v, c_out.at[wid])


def _sc(src, dst, a_edge, a_src, a_dst, n2g):
    mesh = plsc.VectorSubcoreMesh(core_axis_name="c", subcore_axis_name="s")
    f = functools.partial(
        pl.kernel, mesh=mesh,
        compiler_params=pltpu.CompilerParams(needs_layout_passes=False),
        out_type=jax.ShapeDtypeStruct((G, N), jnp.float32),
        scratch_types=[
            pltpu.VMEM((N,), jnp.float32),
            pltpu.VMEM((N,), jnp.float32),
            pltpu.VMEM((N,), jnp.int32),
            pltpu.VMEM((N,), jnp.float32),
            pltpu.VMEM((N,), jnp.float32),
            pltpu.VMEM((CHUNK,), jnp.int32),
            pltpu.VMEM((CHUNK,), jnp.int32),
            pltpu.VMEM((CHUNK,), jnp.float32),
        ],
    )(_sc_body)
    return f(src, dst, a_edge, a_src, a_dst, n2g)


def _tc3_body(c, xt, n2g, b_gat, W_m1, b_m1, W_m2, b_m2, out_o):
    f32 = jnp.float32
    gs = jnp.dot(c[...], xt[...], preferred_element_type=f32)
    gidx = lax.broadcasted_iota(jnp.int32, (N, G), 1)
    oh = (n2g[...] == gidx).astype(f32)
    counts = jnp.sum(oh, axis=0).reshape(G, 1)
    gm = (gs + counts * b_gat[...]) / jnp.maximum(counts, 1.0)
    gm = jnp.maximum(gm, 0.0)
    g1 = jnp.maximum(jnp.dot(gm, W_m1[...], preferred_element_type=f32)
                     + b_m1[...], 0.0)
    out_o[...] = jnp.dot(g1, W_m2[...], preferred_element_type=f32) + b_m2[...]


def _tc3(c, xt, n2g, b_gat, W_m1, b_m1, W_m2, b_m2):
    return pl.pallas_call(
        _tc3_body,
        out_shape=jax.ShapeDtypeStruct((G, 1), jnp.float32),
    )(c, xt, n2g.reshape(N, 1), b_gat.reshape(1, -1),
      W_m1, b_m1.reshape(1, -1), W_m2, b_m2.reshape(1, -1))


def kernel(x, node_ids, node_types, node_scores, edge_index, edge_type,
           edge_attr, node2graph, W_x2h, b_x2h, W_nt, b_nt, W_ns, b_ns,
           W_h2h, b_h2h, W_e1, b_e1, W_e2, b_e2, W_gat, W_gat_e,
           att_src, att_dst, att_edge, b_gat, W_m1, b_m1, W_m2, b_m2):
    src = edge_index[0].astype(jnp.int32)
    dst = edge_index[1].astype(jnp.int32)
    n2g = node2graph.astype(jnp.int32)

    xt, a_s, a_d = _tc1(x, node_types, node_scores, W_x2h, b_x2h, W_nt, b_nt,
                        W_ns, b_ns, W_h2h, b_h2h, W_gat, att_src, att_dst)
    a_e = _tc2(edge_attr, W_e1, b_e1, W_e2, b_e2, W_gat_e, att_edge)
    c = _sc(src, dst, a_e.reshape(E), a_s.reshape(N), a_d.reshape(N), n2g)
    return _tc3(c, xt, n2g, b_gat, W_m1, b_m1, W_m2, b_m2)


# speedup vs baseline: 8.5635x; 1.5480x over previous
import functools
import jax
import jax.numpy as jnp
from jax import lax
from jax.experimental import pallas as pl
from jax.experimental.pallas import tpu as pltpu, tpu_sc as plsc

N = 10000
E = 320000
G = 32
HID = 128
CHUNK = 2000
NSTEP = CHUNK // 16
NCHUNK = E // CHUNK


def _tc1_body(x, nt, ns, W_x2h, b_x2h, W_nt, b_nt, W_ns, b_ns,
              Wh_a, Wh_b, Wh_c, b_h2h, W_gat, att_s, att_d,
              xt_o, as_o, ad_o):
    f32 = jnp.float32
    h1 = jnp.dot(x[...], W_x2h[...], preferred_element_type=f32) + b_x2h[...]
    ntE = jnp.dot(nt[...], W_nt[...], preferred_element_type=f32) + b_nt[...]
    nsE = ns[...] * W_ns[...] + b_ns[...]
    h = (jnp.dot(h1, Wh_a[...], preferred_element_type=f32)
         + jnp.dot(ntE, Wh_b[...], preferred_element_type=f32)
         + jnp.dot(nsE, Wh_c[...], preferred_element_type=f32)
         + b_h2h[...])
    h = jnp.maximum(h, 0.0)
    xt = jnp.dot(h, W_gat[...], preferred_element_type=f32)
    xt_o[...] = xt
    as_o[...] = jnp.sum(xt * att_s[...], axis=1, keepdims=True)
    ad_o[...] = jnp.sum(xt * att_d[...], axis=1, keepdims=True)


def _tc1(x, node_types, node_scores, W_x2h, b_x2h, W_nt, b_nt, W_ns, b_ns,
         W_h2h, b_h2h, W_gat, att_src, att_dst):
    BN = 400
    grid = (N // BN,)
    row = lambda i: (i, 0)
    zero = lambda i: (0, 0)
    in_specs = [
        pl.BlockSpec((BN, 128), row),
        pl.BlockSpec((BN, 4), row),
        pl.BlockSpec((BN, 1), row),
        pl.BlockSpec((128, HID), zero),
        pl.BlockSpec((1, HID), zero),
        pl.BlockSpec((4, HID // 2), zero),
        pl.BlockSpec((1, HID // 2), zero),
        pl.BlockSpec((1, HID // 2), zero),
        pl.BlockSpec((1, HID // 2), zero),
        pl.BlockSpec((HID, HID), zero),
        pl.BlockSpec((HID // 2, HID), zero),
        pl.BlockSpec((HID // 2, HID), zero),
        pl.BlockSpec((1, HID), zero),
        pl.BlockSpec((HID, HID), zero),
        pl.BlockSpec((1, HID), zero),
        pl.BlockSpec((1, HID), zero),
    ]
    out_specs = [
        pl.BlockSpec((BN, HID), row),
        pl.BlockSpec((BN, 1), row),
        pl.BlockSpec((BN, 1), row),
    ]
    out_shape = [
        jax.ShapeDtypeStruct((N, HID), jnp.float32),
        jax.ShapeDtypeStruct((N, 1), jnp.float32),
        jax.ShapeDtypeStruct((N, 1), jnp.float32),
    ]
    return pl.pallas_call(
        _tc1_body, grid=grid, in_specs=in_specs, out_specs=out_specs,
        out_shape=out_shape,
    )(x, node_types, node_scores,
      W_x2h, b_x2h.reshape(1, -1), W_nt, b_nt.reshape(1, -1),
      W_ns, b_ns.reshape(1, -1),
      W_h2h[:HID], W_h2h[HID:HID + HID // 2], W_h2h[HID + HID // 2:],
      b_h2h.reshape(1, -1), W_gat,
      att_src.reshape(1, -1), att_dst.reshape(1, -1))


def _tc2_body(ea, W_e1, b_e1, W_e2, b_e2, W_gat_e, att_e, ae_o):
    f32 = jnp.float32
    h1 = jnp.maximum(jnp.dot(ea[...], W_e1[...], preferred_element_type=f32)
                     + b_e1[...], 0.0)
    h2 = jnp.maximum(jnp.dot(h1, W_e2[...], preferred_element_type=f32)
                     + b_e2[...], 0.0)
    v = jnp.dot(W_gat_e[...], att_e[...].reshape(HID, 1),
                preferred_element_type=f32)
    ae_o[...] = jnp.dot(h2, v, preferred_element_type=f32)


def _tc2(edge_attr, W_e1, b_e1, W_e2, b_e2, W_gat_e, att_edge):
    BE = 3200
    grid = (E // BE,)
    row = lambda i: (i, 0)
    zero = lambda i: (0, 0)
    in_specs = [
        pl.BlockSpec((BE, 46), row),
        pl.BlockSpec((46, HID), zero),
        pl.BlockSpec((1, HID), zero),
        pl.BlockSpec((HID, HID), zero),
        pl.BlockSpec((1, HID), zero),
        pl.BlockSpec((HID, HID), zero),
        pl.BlockSpec((1, HID), zero),
    ]
    return pl.pallas_call(
        _tc2_body, grid=grid, in_specs=in_specs,
        out_specs=pl.BlockSpec((BE, 1), row),
        out_shape=jax.ShapeDtypeStruct((E, 1), jnp.float32),
    )(edge_attr, W_e1, b_e1.reshape(1, -1), W_e2, b_e2.reshape(1, -1),
      W_gat_e, att_edge.reshape(1, -1))


def _edge_vals(src_v, dst_v, ae_v, as_v, ad_v, j):
    s16 = src_v[pl.ds(j * 16, 16)]
    d16 = dst_v[pl.ds(j * 16, 16)]
    ae16 = ae_v[pl.ds(j * 16, 16)]
    a = plsc.load_gather(as_v, [s16]) + plsc.load_gather(ad_v, [d16]) + ae16
    a = jnp.maximum(a, 0.2 * a)
    return s16, d16, jnp.exp(a)


NCHPW = NCHUNK // 32


def _scA_body(src_h, dst_h, ae_h, as_h, ad_h, dp_out,
              as_v, ad_v, den_v, src_v, dst_v, ae_v):
    wid = lax.axis_index("s") * 2 + lax.axis_index("c")

    pltpu.sync_copy(as_h, as_v)
    pltpu.sync_copy(ad_h, ad_v)

    def zero_body(i, carry):
        den_v[pl.ds(i * 16, 16)] = jnp.zeros((16,), jnp.float32)
        return carry
    lax.fori_loop(0, N // 16, zero_body, 0)

    def chunk(k, carry):
        ci = wid * NCHPW + k
        pltpu.sync_copy(src_h.at[pl.ds(ci * CHUNK, CHUNK)], src_v)
        pltpu.sync_copy(dst_h.at[pl.ds(ci * CHUNK, CHUNK)], dst_v)
        pltpu.sync_copy(ae_h.at[pl.ds(ci * CHUNK, CHUNK)], ae_v)

        def step(j, c2):
            _, d16, ex = _edge_vals(src_v, dst_v, ae_v, as_v, ad_v, j)
            plsc.addupdate_scatter(den_v, [d16], ex)
            return c2
        return lax.fori_loop(0, NSTEP, step, carry)
    lax.fori_loop(0, NCHPW, chunk, 0)

    pltpu.sync_copy(den_v, dp_out.at[wid])


def _scB_body(src_h, dst_h, ae_h, as_h, ad_h, n2g_h, den_h, c_out,
              as_v, ad_v, n2g_v, den_v, c_v, src_v, dst_v, ae_v):
    wid = lax.axis_index("s") * 2 + lax.axis_index("c")

    pltpu.sync_copy(as_h, as_v)
    pltpu.sync_copy(ad_h, ad_v)
    pltpu.sync_copy(n2g_h, n2g_v)
    pltpu.sync_copy(den_h, den_v)

    def zero_body(i, carry):
        c_v[pl.ds(i * 16, 16)] = jnp.zeros((16,), jnp.float32)
        return carry
    lax.fori_loop(0, N // 16, zero_body, 0)

    def chunk(ci, carry):
        pltpu.sync_copy(src_h.at[pl.ds(ci * CHUNK, CHUNK)], src_v)
        pltpu.sync_copy(dst_h.at[pl.ds(ci * CHUNK, CHUNK)], dst_v)
        pltpu.sync_copy(ae_h.at[pl.ds(ci * CHUNK, CHUNK)], ae_v)

        def step(j, c2):
            s16, d16, ex = _edge_vals(src_v, dst_v, ae_v, as_v, ad_v, j)
            m = plsc.load_gather(n2g_v, [d16]) == wid
            den = plsc.load_gather(den_v, [d16])
            w = ex / (den + 1e-16)
            plsc.addupdate_scatter(c_v, [s16], w, mask=m)
            return c2
        return lax.fori_loop(0, NSTEP, step, carry)
    lax.fori_loop(0, NCHUNK, chunk, 0)

    pltpu.sync_copy(c_v, c_out.at[wid])


def _dreduce_body(dp, den_o):
    den_o[...] = jnp.sum(dp[...], axis=0, keepdims=True)


def _sc(src, dst, a_edge, a_src, a_dst, n2g):
    mesh = plsc.VectorSubcoreMesh(core_axis_name="c", subcore_axis_name="s")
    cp = pltpu.CompilerParams(needs_layout_passes=False)
    edge_bufs = [
        pltpu.VMEM((CHUNK,), jnp.int32),
        pltpu.VMEM((CHUNK,), jnp.int32),
        pltpu.VMEM((CHUNK,), jnp.float32),
    ]
    dparts = functools.partial(
        pl.kernel, mesh=mesh, compiler_params=cp,
        out_type=jax.ShapeDtypeStruct((32, N), jnp.float32),
        scratch_types=[
            pltpu.VMEM((N,), jnp.float32),
            pltpu.VMEM((N,), jnp.float32),
            pltpu.VMEM((N,), jnp.float32),
        ] + edge_bufs,
    )(_scA_body)(src, dst, a_edge, a_src, a_dst)

    denom = pl.pallas_call(
        _dreduce_body,
        out_shape=jax.ShapeDtypeStruct((1, N), jnp.float32),
    )(dparts).reshape(N)

    return functools.partial(
        pl.kernel, mesh=mesh, compiler_params=cp,
        out_type=jax.ShapeDtypeStruct((G, N), jnp.float32),
        scratch_types=[
            pltpu.VMEM((N,), jnp.float32),
            pltpu.VMEM((N,), jnp.float32),
            pltpu.VMEM((N,), jnp.int32),
            pltpu.VMEM((N,), jnp.float32),
            pltpu.VMEM((N,), jnp.float32),
        ] + edge_bufs,
    )(_scB_body)(src, dst, a_edge, a_src, a_dst, n2g, denom)


def _tc3_body(c, xt, n2g, b_gat, W_m1, b_m1, W_m2, b_m2, out_o):
    f32 = jnp.float32
    gs = jnp.dot(c[...], xt[...], preferred_element_type=f32)
    gidx = lax.broadcasted_iota(jnp.int32, (N, G), 1)
    oh = (n2g[...] == gidx).astype(f32)
    counts = jnp.sum(oh, axis=0).reshape(G, 1)
    gm = (gs + counts * b_gat[...]) / jnp.maximum(counts, 1.0)
    gm = jnp.maximum(gm, 0.0)
    g1 = jnp.maximum(jnp.dot(gm, W_m1[...], preferred_element_type=f32)
                     + b_m1[...], 0.0)
    out_o[...] = jnp.dot(g1, W_m2[...], preferred_element_type=f32) + b_m2[...]


def _tc3(c, xt, n2g, b_gat, W_m1, b_m1, W_m2, b_m2):
    return pl.pallas_call(
        _tc3_body,
        out_shape=jax.ShapeDtypeStruct((G, 1), jnp.float32),
    )(c, xt, n2g.reshape(N, 1), b_gat.reshape(1, -1),
      W_m1, b_m1.reshape(1, -1), W_m2, b_m2.reshape(1, -1))


def kernel(x, node_ids, node_types, node_scores, edge_index, edge_type,
           edge_attr, node2graph, W_x2h, b_x2h, W_nt, b_nt, W_ns, b_ns,
           W_h2h, b_h2h, W_e1, b_e1, W_e2, b_e2, W_gat, W_gat_e,
           att_src, att_dst, att_edge, b_gat, W_m1, b_m1, W_m2, b_m2):
    src = edge_index[0].astype(jnp.int32)
    dst = edge_index[1].astype(jnp.int32)
    n2g = node2graph.astype(jnp.int32)

    xt, a_s, a_d = _tc1(x, node_types, node_scores, W_x2h, b_x2h, W_nt, b_nt,
                        W_ns, b_ns, W_h2h, b_h2h, W_gat, att_src, att_dst)
    a_e = _tc2(edge_attr, W_e1, b_e1, W_e2, b_e2, W_gat_e, att_edge)
    c = _sc(src, dst, a_e.reshape(E), a_s.reshape(N), a_d.reshape(N), n2g)
    return _tc3(c, xt, n2g, b_gat, W_m1, b_m1, W_m2, b_m2)


# speedup vs baseline: 10.1683x; 1.1874x over previous
import functools
import jax
import jax.numpy as jnp
from jax import lax
from jax.experimental import pallas as pl
from jax.experimental.pallas import tpu as pltpu, tpu_sc as plsc

N = 10000
E = 320000
G = 32
HID = 128
CHUNK = 2000
NSTEP = CHUNK // 16
NCHUNK = E // CHUNK


def _tc1_body(x, nt, ns, W_x2h, b_x2h, W_nt, b_nt, W_ns, b_ns,
              Wh_a, Wh_b, Wh_c, b_h2h, W_gat, att_s, att_d,
              xt_o, as_o, ad_o):
    f32 = jnp.float32
    h1 = jnp.dot(x[...], W_x2h[...], preferred_element_type=f32) + b_x2h[...]
    ntE = jnp.dot(nt[...], W_nt[...], preferred_element_type=f32) + b_nt[...]
    nsE = ns[...] * W_ns[...] + b_ns[...]
    h = (jnp.dot(h1, Wh_a[...], preferred_element_type=f32)
         + jnp.dot(ntE, Wh_b[...], preferred_element_type=f32)
         + jnp.dot(nsE, Wh_c[...], preferred_element_type=f32)
         + b_h2h[...])
    h = jnp.maximum(h, 0.0)
    xt = jnp.dot(h, W_gat[...], preferred_element_type=f32)
    xt_o[...] = xt
    as_o[...] = jnp.sum(xt * att_s[...], axis=1, keepdims=True)
    ad_o[...] = jnp.sum(xt * att_d[...], axis=1, keepdims=True)


def _tc1(x, node_types, node_scores, W_x2h, b_x2h, W_nt, b_nt, W_ns, b_ns,
         W_h2h, b_h2h, W_gat, att_src, att_dst):
    BN = 400
    grid = (N // BN,)
    row = lambda i: (i, 0)
    zero = lambda i: (0, 0)
    in_specs = [
        pl.BlockSpec((BN, 128), row),
        pl.BlockSpec((BN, 4), row),
        pl.BlockSpec((BN, 1), row),
        pl.BlockSpec((128, HID), zero),
        pl.BlockSpec((1, HID), zero),
        pl.BlockSpec((4, HID // 2), zero),
        pl.BlockSpec((1, HID // 2), zero),
        pl.BlockSpec((1, HID // 2), zero),
        pl.BlockSpec((1, HID // 2), zero),
        pl.BlockSpec((HID, HID), zero),
        pl.BlockSpec((HID // 2, HID), zero),
        pl.BlockSpec((HID // 2, HID), zero),
        pl.BlockSpec((1, HID), zero),
        pl.BlockSpec((HID, HID), zero),
        pl.BlockSpec((1, HID), zero),
        pl.BlockSpec((1, HID), zero),
    ]
    out_specs = [
        pl.BlockSpec((BN, HID), row),
        pl.BlockSpec((BN, 1), row),
        pl.BlockSpec((BN, 1), row),
    ]
    out_shape = [
        jax.ShapeDtypeStruct((N, HID), jnp.float32),
        jax.ShapeDtypeStruct((N, 1), jnp.float32),
        jax.ShapeDtypeStruct((N, 1), jnp.float32),
    ]
    return pl.pallas_call(
        _tc1_body, grid=grid, in_specs=in_specs, out_specs=out_specs,
        out_shape=out_shape,
    )(x, node_types, node_scores,
      W_x2h, b_x2h.reshape(1, -1), W_nt, b_nt.reshape(1, -1),
      W_ns, b_ns.reshape(1, -1),
      W_h2h[:HID], W_h2h[HID:HID + HID // 2], W_h2h[HID + HID // 2:],
      b_h2h.reshape(1, -1), W_gat,
      att_src.reshape(1, -1), att_dst.reshape(1, -1))


def _tc2_body(ea, W_e1, b_e1, W_e2, b_e2, W_gat_e, att_e, ae_o):
    f32 = jnp.float32
    h1 = jnp.maximum(jnp.dot(ea[...], W_e1[...], preferred_element_type=f32)
                     + b_e1[...], 0.0)
    h2 = jnp.maximum(jnp.dot(h1, W_e2[...], preferred_element_type=f32)
                     + b_e2[...], 0.0)
    v = jnp.dot(W_gat_e[...], att_e[...].reshape(HID, 1),
                preferred_element_type=f32)
    ae_o[...] = jnp.dot(h2, v, preferred_element_type=f32)


def _tc2(edge_attr, W_e1, b_e1, W_e2, b_e2, W_gat_e, att_edge):
    BE = 3200
    grid = (E // BE,)
    row = lambda i: (i, 0)
    zero = lambda i: (0, 0)
    in_specs = [
        pl.BlockSpec((BE, 46), row),
        pl.BlockSpec((46, HID), zero),
        pl.BlockSpec((1, HID), zero),
        pl.BlockSpec((HID, HID), zero),
        pl.BlockSpec((1, HID), zero),
        pl.BlockSpec((HID, HID), zero),
        pl.BlockSpec((1, HID), zero),
    ]
    return pl.pallas_call(
        _tc2_body, grid=grid, in_specs=in_specs,
        out_specs=pl.BlockSpec((BE, 1), row),
        out_shape=jax.ShapeDtypeStruct((E, 1), jnp.float32),
    )(edge_attr, W_e1, b_e1.reshape(1, -1), W_e2, b_e2.reshape(1, -1),
      W_gat_e, att_edge.reshape(1, -1))


def _edge_vals(src_v, dst_v, ae_v, as_v, ad_v, j):
    s16 = src_v[pl.ds(j * 16, 16)]
    d16 = dst_v[pl.ds(j * 16, 16)]
    ae16 = ae_v[pl.ds(j * 16, 16)]
    a = plsc.load_gather(as_v, [s16]) + plsc.load_gather(ad_v, [d16]) + ae16
    a = jnp.maximum(a, 0.2 * a)
    return s16, d16, jnp.exp(a)


NCHPW = NCHUNK // 32


def _scA_body(src_h, dst_h, ae_h, as_h, ad_h, dp_out, ex_out,
              as_v, ad_v, den_v, src_v, dst_v, ae_v, ex_v):
    wid = lax.axis_index("s") * 2 + lax.axis_index("c")

    pltpu.sync_copy(as_h, as_v)
    pltpu.sync_copy(ad_h, ad_v)

    def zero_body(i, carry):
        den_v[pl.ds(i * 16, 16)] = jnp.zeros((16,), jnp.float32)
        return carry
    lax.fori_loop(0, N // 16, zero_body, 0)

    def chunk(k, carry):
        ci = wid * NCHPW + k
        pltpu.sync_copy(src_h.at[pl.ds(ci * CHUNK, CHUNK)], src_v)
        pltpu.sync_copy(dst_h.at[pl.ds(ci * CHUNK, CHUNK)], dst_v)
        pltpu.sync_copy(ae_h.at[pl.ds(ci * CHUNK, CHUNK)], ae_v)

        def step(j, c2):
            _, d16, ex = _edge_vals(src_v, dst_v, ae_v, as_v, ad_v, j)
            plsc.addupdate_scatter(den_v, [d16], ex)
            ex_v[pl.ds(j * 16, 16)] = ex
            return c2
        r = lax.fori_loop(0, NSTEP, step, carry)
        pltpu.sync_copy(ex_v, ex_out.at[pl.ds(ci * CHUNK, CHUNK)])
        return r
    lax.fori_loop(0, NCHPW, chunk, 0)

    pltpu.sync_copy(den_v, dp_out.at[wid])


def _scB_body(src_h, dst_h, ex_h, n2g_h, rden_h, c_out,
              n2g_v, rden_v, c_v, src_v, dst_v, ex_v):
    wid = lax.axis_index("s") * 2 + lax.axis_index("c")

    pltpu.sync_copy(n2g_h, n2g_v)
    pltpu.sync_copy(rden_h, rden_v)

    def zero_body(i, carry):
        c_v[pl.ds(i * 16, 16)] = jnp.zeros((16,), jnp.float32)
        return carry
    lax.fori_loop(0, N // 16, zero_body, 0)

    def chunk(ci, carry):
        pltpu.sync_copy(src_h.at[pl.ds(ci * CHUNK, CHUNK)], src_v)
        pltpu.sync_copy(dst_h.at[pl.ds(ci * CHUNK, CHUNK)], dst_v)
        pltpu.sync_copy(ex_h.at[pl.ds(ci * CHUNK, CHUNK)], ex_v)

        def step(j, c2):
            s16 = src_v[pl.ds(j * 16, 16)]
            d16 = dst_v[pl.ds(j * 16, 16)]
            ex16 = ex_v[pl.ds(j * 16, 16)]
            m = plsc.load_gather(n2g_v, [d16]) == wid
            w = ex16 * plsc.load_gather(rden_v, [d16])
            plsc.addupdate_scatter(c_v, [s16], w, mask=m)
            return c2
        return lax.fori_loop(0, NSTEP, step, carry)
    lax.fori_loop(0, NCHUNK, chunk, 0)

    pltpu.sync_copy(c_v, c_out.at[wid])


def _dreduce_body(dp, rden_o):
    rden_o[...] = 1.0 / (jnp.sum(dp[...], axis=0, keepdims=True) + 1e-16)


def _sc(src, dst, a_edge, a_src, a_dst, n2g):
    mesh = plsc.VectorSubcoreMesh(core_axis_name="c", subcore_axis_name="s")
    cp = pltpu.CompilerParams(needs_layout_passes=False)
    edge_bufs = [
        pltpu.VMEM((CHUNK,), jnp.int32),
        pltpu.VMEM((CHUNK,), jnp.int32),
        pltpu.VMEM((CHUNK,), jnp.float32),
    ]
    dparts, ex = functools.partial(
        pl.kernel, mesh=mesh, compiler_params=cp,
        out_type=[
            jax.ShapeDtypeStruct((32, N), jnp.float32),
            jax.ShapeDtypeStruct((E,), jnp.float32),
        ],
        scratch_types=[
            pltpu.VMEM((N,), jnp.float32),
            pltpu.VMEM((N,), jnp.float32),
            pltpu.VMEM((N,), jnp.float32),
        ] + edge_bufs + [pltpu.VMEM((CHUNK,), jnp.float32)],
    )(_scA_body)(src, dst, a_edge, a_src, a_dst)

    rden = pl.pallas_call(
        _dreduce_body,
        out_shape=jax.ShapeDtypeStruct((1, N), jnp.float32),
    )(dparts).reshape(N)

    return functools.partial(
        pl.kernel, mesh=mesh, compiler_params=cp,
        out_type=jax.ShapeDtypeStruct((G, N), jnp.float32),
        scratch_types=[
            pltpu.VMEM((N,), jnp.int32),
            pltpu.VMEM((N,), jnp.float32),
            pltpu.VMEM((N,), jnp.float32),
        ] + edge_bufs[:2] + [pltpu.VMEM((CHUNK,), jnp.float32)],
    )(_scB_body)(src, dst, ex, n2g, rden)


def _tc3_body(c, xt, n2g, b_gat, W_m1, b_m1, W_m2, b_m2, out_o):
    f32 = jnp.float32
    gs = jnp.dot(c[...], xt[...], preferred_element_type=f32)
    gidx = lax.broadcasted_iota(jnp.int32, (N, G), 1)
    oh = (n2g[...] == gidx).astype(f32)
    counts = jnp.sum(oh, axis=0).reshape(G, 1)
    gm = (gs + counts * b_gat[...]) / jnp.maximum(counts, 1.0)
    gm = jnp.maximum(gm, 0.0)
    g1 = jnp.maximum(jnp.dot(gm, W_m1[...], preferred_element_type=f32)
                     + b_m1[...], 0.0)
    out_o[...] = jnp.dot(g1, W_m2[...], preferred_element_type=f32) + b_m2[...]


def _tc3(c, xt, n2g, b_gat, W_m1, b_m1, W_m2, b_m2):
    return pl.pallas_call(
        _tc3_body,
        out_shape=jax.ShapeDtypeStruct((G, 1), jnp.float32),
    )(c, xt, n2g.reshape(N, 1), b_gat.reshape(1, -1),
      W_m1, b_m1.reshape(1, -1), W_m2, b_m2.reshape(1, -1))


def kernel(x, node_ids, node_types, node_scores, edge_index, edge_type,
           edge_attr, node2graph, W_x2h, b_x2h, W_nt, b_nt, W_ns, b_ns,
           W_h2h, b_h2h, W_e1, b_e1, W_e2, b_e2, W_gat, W_gat_e,
           att_src, att_dst, att_edge, b_gat, W_m1, b_m1, W_m2, b_m2):
    src = edge_index[0].astype(jnp.int32)
    dst = edge_index[1].astype(jnp.int32)
    n2g = node2graph.astype(jnp.int32)

    xt, a_s, a_d = _tc1(x, node_types, node_scores, W_x2h, b_x2h, W_nt, b_nt,
                        W_ns, b_ns, W_h2h, b_h2h, W_gat, att_src, att_dst)
    a_e = _tc2(edge_attr, W_e1, b_e1, W_e2, b_e2, W_gat_e, att_edge)
    c = _sc(src, dst, a_e.reshape(E), a_s.reshape(N), a_d.reshape(N), n2g)
    return _tc3(c, xt, n2g, b_gat, W_m1, b_m1, W_m2, b_m2)


# speedup vs baseline: 12.8674x; 1.2654x over previous
import functools
import jax
import jax.numpy as jnp
from jax import lax
from jax.experimental import pallas as pl
from jax.experimental.pallas import tpu as pltpu, tpu_sc as plsc

N = 10000
E = 320000
G = 32
HID = 128
CHUNK = 10000
NSTEP = CHUNK // 16
NCHUNK = E // CHUNK


def _tc1_body(x, nt, ns, W_x2h, b_x2h, W_nt, b_nt, W_ns, b_ns,
              Wh_a, Wh_b, Wh_c, b_h2h, W_gat, att_s, att_d,
              xt_o, as_o, ad_o):
    f32 = jnp.float32
    h1 = jnp.dot(x[...], W_x2h[...], preferred_element_type=f32) + b_x2h[...]
    ntE = jnp.dot(nt[...], W_nt[...], preferred_element_type=f32) + b_nt[...]
    nsE = ns[...] * W_ns[...] + b_ns[...]
    h = (jnp.dot(h1, Wh_a[...], preferred_element_type=f32)
         + jnp.dot(ntE, Wh_b[...], preferred_element_type=f32)
         + jnp.dot(nsE, Wh_c[...], preferred_element_type=f32)
         + b_h2h[...])
    h = jnp.maximum(h, 0.0)
    xt = jnp.dot(h, W_gat[...], preferred_element_type=f32)
    xt_o[...] = xt
    as_o[...] = jnp.sum(xt * att_s[...], axis=1, keepdims=True)
    ad_o[...] = jnp.sum(xt * att_d[...], axis=1, keepdims=True)


def _tc1(x, node_types, node_scores, W_x2h, b_x2h, W_nt, b_nt, W_ns, b_ns,
         W_h2h, b_h2h, W_gat, att_src, att_dst):
    BN = 400
    grid = (N // BN,)
    row = lambda i: (i, 0)
    zero = lambda i: (0, 0)
    in_specs = [
        pl.BlockSpec((BN, 128), row),
        pl.BlockSpec((BN, 4), row),
        pl.BlockSpec((BN, 1), row),
        pl.BlockSpec((128, HID), zero),
        pl.BlockSpec((1, HID), zero),
        pl.BlockSpec((4, HID // 2), zero),
        pl.BlockSpec((1, HID // 2), zero),
        pl.BlockSpec((1, HID // 2), zero),
        pl.BlockSpec((1, HID // 2), zero),
        pl.BlockSpec((HID, HID), zero),
        pl.BlockSpec((HID // 2, HID), zero),
        pl.BlockSpec((HID // 2, HID), zero),
        pl.BlockSpec((1, HID), zero),
        pl.BlockSpec((HID, HID), zero),
        pl.BlockSpec((1, HID), zero),
        pl.BlockSpec((1, HID), zero),
    ]
    out_specs = [
        pl.BlockSpec((BN, HID), row),
        pl.BlockSpec((BN, 1), row),
        pl.BlockSpec((BN, 1), row),
    ]
    out_shape = [
        jax.ShapeDtypeStruct((N, HID), jnp.float32),
        jax.ShapeDtypeStruct((N, 1), jnp.float32),
        jax.ShapeDtypeStruct((N, 1), jnp.float32),
    ]
    return pl.pallas_call(
        _tc1_body, grid=grid, in_specs=in_specs, out_specs=out_specs,
        out_shape=out_shape,
    )(x, node_types, node_scores,
      W_x2h, b_x2h.reshape(1, -1), W_nt, b_nt.reshape(1, -1),
      W_ns, b_ns.reshape(1, -1),
      W_h2h[:HID], W_h2h[HID:HID + HID // 2], W_h2h[HID + HID // 2:],
      b_h2h.reshape(1, -1), W_gat,
      att_src.reshape(1, -1), att_dst.reshape(1, -1))


def _tc2_body(ea, W_e1, b_e1, W_e2, b_e2, W_gat_e, att_e, ae_o):
    f32 = jnp.float32
    h1 = jnp.maximum(jnp.dot(ea[...], W_e1[...], preferred_element_type=f32)
                     + b_e1[...], 0.0)
    h2 = jnp.maximum(jnp.dot(h1, W_e2[...], preferred_element_type=f32)
                     + b_e2[...], 0.0)
    v = jnp.dot(W_gat_e[...], att_e[...].reshape(HID, 1),
                preferred_element_type=f32)
    ae_o[...] = jnp.dot(h2, v, preferred_element_type=f32)


def _tc2(edge_attr, W_e1, b_e1, W_e2, b_e2, W_gat_e, att_edge):
    BE = 3200
    grid = (E // BE,)
    row = lambda i: (i, 0)
    zero = lambda i: (0, 0)
    in_specs = [
        pl.BlockSpec((BE, 46), row),
        pl.BlockSpec((46, HID), zero),
        pl.BlockSpec((1, HID), zero),
        pl.BlockSpec((HID, HID), zero),
        pl.BlockSpec((1, HID), zero),
        pl.BlockSpec((HID, HID), zero),
        pl.BlockSpec((1, HID), zero),
    ]
    return pl.pallas_call(
        _tc2_body, grid=grid, in_specs=in_specs,
        out_specs=pl.BlockSpec((BE, 1), row),
        out_shape=jax.ShapeDtypeStruct((E, 1), jnp.float32),
    )(edge_attr, W_e1, b_e1.reshape(1, -1), W_e2, b_e2.reshape(1, -1),
      W_gat_e, att_edge.reshape(1, -1))


def _edge_vals(src_v, dst_v, ae_v, as_v, ad_v, j):
    s16 = src_v[pl.ds(j * 16, 16)]
    d16 = dst_v[pl.ds(j * 16, 16)]
    ae16 = ae_v[pl.ds(j * 16, 16)]
    a = plsc.load_gather(as_v, [s16]) + plsc.load_gather(ad_v, [d16]) + ae16
    a = jnp.maximum(a, 0.2 * a)
    return s16, d16, jnp.exp(a)


NCHPW = NCHUNK // 32


def _scA_body(src_h, dst_h, ae_h, as_h, ad_h, dp_out, ex_out,
              as_v, ad_v, den_v, src_v, dst_v, ae_v, ex_v):
    wid = lax.axis_index("s") * 2 + lax.axis_index("c")

    pltpu.sync_copy(as_h, as_v)
    pltpu.sync_copy(ad_h, ad_v)

    def zero_body(i, carry):
        den_v[pl.ds(i * 16, 16)] = jnp.zeros((16,), jnp.float32)
        return carry
    lax.fori_loop(0, N // 16, zero_body, 0)

    def chunk(k, carry):
        ci = wid * NCHPW + k
        pltpu.sync_copy(src_h.at[pl.ds(ci * CHUNK, CHUNK)], src_v)
        pltpu.sync_copy(dst_h.at[pl.ds(ci * CHUNK, CHUNK)], dst_v)
        pltpu.sync_copy(ae_h.at[pl.ds(ci * CHUNK, CHUNK)], ae_v)

        def step(j, c2):
            _, d16, ex = _edge_vals(src_v, dst_v, ae_v, as_v, ad_v, j)
            plsc.addupdate_scatter(den_v, [d16], ex)
            ex_v[pl.ds(j * 16, 16)] = ex
            return c2
        r = lax.fori_loop(0, NSTEP, step, carry)
        pltpu.sync_copy(ex_v, ex_out.at[pl.ds(ci * CHUNK, CHUNK)])
        return r
    lax.fori_loop(0, NCHPW, chunk, 0)

    pltpu.sync_copy(den_v, dp_out.at[wid])


def _scB_body(src_h, dst_h, ex_h, n2g_h, rden_h, c_out,
              n2g_v, rden_v, c_v, src_v, dst_v, ex_v):
    wid = lax.axis_index("s") * 2 + lax.axis_index("c")

    pltpu.sync_copy(n2g_h, n2g_v)
    pltpu.sync_copy(rden_h, rden_v)

    def zero_body(i, carry):
        c_v[pl.ds(i * 16, 16)] = jnp.zeros((16,), jnp.float32)
        return carry
    lax.fori_loop(0, N // 16, zero_body, 0)

    def chunk(ci, carry):
        pltpu.sync_copy(src_h.at[pl.ds(ci * CHUNK, CHUNK)], src_v)
        pltpu.sync_copy(dst_h.at[pl.ds(ci * CHUNK, CHUNK)], dst_v)
        pltpu.sync_copy(ex_h.at[pl.ds(ci * CHUNK, CHUNK)], ex_v)

        def step(j, c2):
            s16 = src_v[pl.ds(j * 16, 16)]
            d16 = dst_v[pl.ds(j * 16, 16)]
            ex16 = ex_v[pl.ds(j * 16, 16)]
            m = plsc.load_gather(n2g_v, [d16]) == wid
            w = ex16 * plsc.load_gather(rden_v, [d16])
            plsc.addupdate_scatter(c_v, [s16], w, mask=m)
            return c2
        return lax.fori_loop(0, NSTEP, step, carry)
    lax.fori_loop(0, NCHUNK, chunk, 0)

    pltpu.sync_copy(c_v, c_out.at[wid])


def _dreduce_body(dp, rden_o):
    rden_o[...] = 1.0 / (jnp.sum(dp[...], axis=0, keepdims=True) + 1e-16)


def _sc(src, dst, a_edge, a_src, a_dst, n2g):
    mesh = plsc.VectorSubcoreMesh(core_axis_name="c", subcore_axis_name="s")
    cp = pltpu.CompilerParams(needs_layout_passes=False)
    edge_bufs = [
        pltpu.VMEM((CHUNK,), jnp.int32),
        pltpu.VMEM((CHUNK,), jnp.int32),
        pltpu.VMEM((CHUNK,), jnp.float32),
    ]
    dparts, ex = functools.partial(
        pl.kernel, mesh=mesh, compiler_params=cp,
        out_type=[
            jax.ShapeDtypeStruct((32, N), jnp.float32),
            jax.ShapeDtypeStruct((E,), jnp.float32),
        ],
        scratch_types=[
            pltpu.VMEM((N,), jnp.float32),
            pltpu.VMEM((N,), jnp.float32),
            pltpu.VMEM((N,), jnp.float32),
        ] + edge_bufs + [pltpu.VMEM((CHUNK,), jnp.float32)],
    )(_scA_body)(src, dst, a_edge, a_src, a_dst)

    rden = pl.pallas_call(
        _dreduce_body,
        out_shape=jax.ShapeDtypeStruct((1, N), jnp.float32),
    )(dparts).reshape(N)

    return functools.partial(
        pl.kernel, mesh=mesh, compiler_params=cp,
        out_type=jax.ShapeDtypeStruct((G, N), jnp.float32),
        scratch_types=[
            pltpu.VMEM((N,), jnp.int32),
            pltpu.VMEM((N,), jnp.float32),
            pltpu.VMEM((N,), jnp.float32),
        ] + edge_bufs[:2] + [pltpu.VMEM((CHUNK,), jnp.float32)],
    )(_scB_body)(src, dst, ex, n2g, rden)


def _tc3_body(c, xt, n2g, b_gat, W_m1, b_m1, W_m2, b_m2, out_o):
    f32 = jnp.float32
    gs = jnp.dot(c[...], xt[...], preferred_element_type=f32)
    gidx = lax.broadcasted_iota(jnp.int32, (N, G), 1)
    oh = (n2g[...] == gidx).astype(f32)
    counts = jnp.sum(oh, axis=0).reshape(G, 1)
    gm = (gs + counts * b_gat[...]) / jnp.maximum(counts, 1.0)
    gm = jnp.maximum(gm, 0.0)
    g1 = jnp.maximum(jnp.dot(gm, W_m1[...], preferred_element_type=f32)
                     + b_m1[...], 0.0)
    out_o[...] = jnp.dot(g1, W_m2[...], preferred_element_type=f32) + b_m2[...]


def _tc3(c, xt, n2g, b_gat, W_m1, b_m1, W_m2, b_m2):
    return pl.pallas_call(
        _tc3_body,
        out_shape=jax.ShapeDtypeStruct((G, 1), jnp.float32),
    )(c, xt, n2g.reshape(N, 1), b_gat.reshape(1, -1),
      W_m1, b_m1.reshape(1, -1), W_m2, b_m2.reshape(1, -1))


def kernel(x, node_ids, node_types, node_scores, edge_index, edge_type,
           edge_attr, node2graph, W_x2h, b_x2h, W_nt, b_nt, W_ns, b_ns,
           W_h2h, b_h2h, W_e1, b_e1, W_e2, b_e2, W_gat, W_gat_e,
           att_src, att_dst, att_edge, b_gat, W_m1, b_m1, W_m2, b_m2):
    src = edge_index[0].astype(jnp.int32)
    dst = edge_index[1].astype(jnp.int32)
    n2g = node2graph.astype(jnp.int32)

    xt, a_s, a_d = _tc1(x, node_types, node_scores, W_x2h, b_x2h, W_nt, b_nt,
                        W_ns, b_ns, W_h2h, b_h2h, W_gat, att_src, att_dst)
    a_e = _tc2(edge_attr, W_e1, b_e1, W_e2, b_e2, W_gat_e, att_edge)
    c = _sc(src, dst, a_e.reshape(E), a_s.reshape(N), a_d.reshape(N), n2g)
    return _tc3(c, xt, n2g, b_gat, W_m1, b_m1, W_m2, b_m2)
